# Initial kernel scaffold; baseline (speedup 1.0000x reference)
import jax
import jax.numpy as jnp
from jax.experimental import pallas as pl


def kernel(x, W_fc1, b_fc1, g1, be1, W_gc, b_gc, g2, be2, W_fc2, b_fc2, g3, be3):
    raise NotImplementedError("write your pallas kernel here")


# speedup vs baseline: 10.5451x; 10.5451x over previous
import functools

import jax
import jax.numpy as jnp
from jax import lax
from jax.experimental import pallas as pl

_K = 9
_KPAD = 16


def _erf(x):
    a1, a2, a3, a4, a5 = 0.254829592, -0.284496736, 1.421413741, -1.453152027, 1.061405429
    p = 0.3275911
    ax = jnp.abs(x)
    t = 1.0 / (1.0 + p * ax)
    poly = ((((a5 * t + a4) * t + a3) * t + a2) * t + a1) * t
    y = 1.0 - poly * jnp.exp(-ax * ax)
    return jnp.sign(x) * y


def _p1(xT_ref, w1t_ref, b1_ref, y_ref, s_ref, ss_ref):
    xT = xT_ref[0]
    y = jnp.dot(xT, w1t_ref[...], preferred_element_type=jnp.float32) + b1_ref[...]
    y_ref[0] = y
    s_ref[0, 0] = jnp.sum(y, axis=0)
    ss_ref[0, 0] = jnp.sum(y * y, axis=0)


def _p2(y_ref, sc_ref, sh_ref, wdT_ref, wjT_ref, idx_ref, aT_ref, bfT_ref, *, TN):
    t = pl.program_id(1)
    N = y_ref.shape[1]
    y = y_ref[0]
    h = y * sc_ref[...] + sh_ref[...]
    nrm = jnp.sqrt(jnp.sum(h * h, axis=1, keepdims=True))
    xn = h / (nrm + 1e-12)
    x2 = jnp.sum(xn * xn, axis=1, keepdims=True)

    h_t = y_ref[0, pl.ds(t * TN, TN), :] * sc_ref[...] + sh_ref[...]
    nrm_t = jnp.sqrt(jnp.sum(h_t * h_t, axis=1, keepdims=True))
    xn_t = h_t / (nrm_t + 1e-12)

    Laug = jnp.concatenate([xn_t, jnp.ones((TN, 1), jnp.float32)], axis=1)
    Raug = jnp.concatenate([2.0 * xn, -x2], axis=1)
    score = lax.dot_general(Laug, Raug, (((1,), (1,)), ((), ())),
                            preferred_element_type=jnp.float32)

    cols = lax.broadcasted_iota(jnp.int32, (TN, N), 1)
    lanek = lax.broadcasted_iota(jnp.int32, (TN, _KPAD), 1)
    acc = jnp.zeros((TN, _KPAD), jnp.int32)
    vals = score
    for k in range(_K):
        rmax = jnp.max(vals, axis=1, keepdims=True)
        am = jnp.min(jnp.where(vals == rmax, cols, jnp.int32(N)), axis=1)
        acc = jnp.where(lanek == k, am[:, None], acc)
        vals = jnp.where(cols == am[:, None], -jnp.inf, vals)
    idx_ref[0] = acc
    aT_ref[0] = lax.dot_general(h_t, wdT_ref[...], (((1,), (0,)), ((), ())),
                                preferred_element_type=jnp.float32)
    bfT_ref[0] = lax.dot_general(h_t, wjT_ref[...], (((1,), (0,)), ((), ())),
                                 preferred_element_type=jnp.float32)


def _p3(idx_ref, bfT_ref, aT_ref, bgc_ref, agg_ref, s_ref, ss_ref):
    TN, N = idx_ref.shape[1], bfT_ref.shape[1]
    bf = bfT_ref[0]
    idxb = idx_ref[0]
    colm = lax.broadcasted_iota(jnp.int32, (TN, N), 1)
    g = None
    for k in range(_K):
        am = idxb[:, k]
        oh = jnp.where(colm == am[:, None], 1.0, 0.0)
        gk = jnp.dot(oh, bf, preferred_element_type=jnp.float32)
        g = gk if g is None else jnp.maximum(g, gk)
    agg = aT_ref[0] + g + bgc_ref[...]
    agg_ref[0] = agg
    s_ref[0, 0] = jnp.sum(agg, axis=0)
    ss_ref[0, 0] = jnp.sum(agg * agg, axis=0)


def _p5(agg_ref, sc_ref, sh_ref, w2t_ref, b2_ref, outp_ref, s_ref, ss_ref):
    z = agg_ref[0] * sc_ref[...] + sh_ref[...]
    gx = 0.5 * z * (1.0 + _erf(z * 0.7071067811865476))
    o = jnp.dot(gx, w2t_ref[...], preferred_element_type=jnp.float32) + b2_ref[...]
    outp_ref[0] = o
    s_ref[0, 0] = jnp.sum(o, axis=0)
    ss_ref[0, 0] = jnp.sum(o * o, axis=0)


def _p6(outp_ref, sc_ref, sh_ref, xT_ref, out_ref):
    out_ref[0] = outp_ref[0] * sc_ref[...] + sh_ref[...] + xT_ref[0]


def _affine(s_p, ss_p, g, be, cnt):
    m = jnp.sum(s_p, axis=(0, 1)) / cnt
    v = jnp.sum(ss_p, axis=(0, 1)) / cnt - m * m
    sc = g / jnp.sqrt(v + 1e-5)
    sh = be - m * sc
    return sc[None, :], sh[None, :]


def kernel(x, W_fc1, b_fc1, g1, be1, W_gc, b_gc, g2, be2, W_fc2, b_fc2, g3, be3):
    B, C, H, Wd = x.shape
    N = H * Wd
    HID = W_gc.shape[0]
    TN = 256 if N % 256 == 0 else N
    NT = N // TN
    f32 = jnp.float32

    xT = jnp.transpose(x.reshape(B, C, N), (0, 2, 1))
    w1t = W_fc1.T
    wi, wj = W_gc[:, :C], W_gc[:, C:]
    wdT = (wi - wj).T
    wjT = wj.T
    w2t = W_fc2.T

    yT, s1p, ss1p = pl.pallas_call(
        _p1,
        grid=(B,),
        in_specs=[
            pl.BlockSpec((1, N, C), lambda b: (b, 0, 0)),
            pl.BlockSpec((C, C), lambda b: (0, 0)),
            pl.BlockSpec((1, C), lambda b: (0, 0)),
        ],
        out_specs=[
            pl.BlockSpec((1, N, C), lambda b: (b, 0, 0)),
            pl.BlockSpec((1, 1, C), lambda b: (b, 0, 0)),
            pl.BlockSpec((1, 1, C), lambda b: (b, 0, 0)),
        ],
        out_shape=[
            jax.ShapeDtypeStruct((B, N, C), f32),
            jax.ShapeDtypeStruct((B, 1, C), f32),
            jax.ShapeDtypeStruct((B, 1, C), f32),
        ],
    )(xT, w1t, b_fc1[None, :])
    sc1, sh1 = _affine(s1p, ss1p, g1, be1, B * N)

    idx, aT, bfT = pl.pallas_call(
        functools.partial(_p2, TN=TN),
        grid=(B, NT),
        in_specs=[
            pl.BlockSpec((1, N, C), lambda b, t: (b, 0, 0)),
            pl.BlockSpec((1, C), lambda b, t: (0, 0)),
            pl.BlockSpec((1, C), lambda b, t: (0, 0)),
            pl.BlockSpec((C, HID), lambda b, t: (0, 0)),
            pl.BlockSpec((C, HID), lambda b, t: (0, 0)),
        ],
        out_specs=[
            pl.BlockSpec((1, TN, _KPAD), lambda b, t: (b, t, 0)),
            pl.BlockSpec((1, TN, HID), lambda b, t: (b, t, 0)),
            pl.BlockSpec((1, TN, HID), lambda b, t: (b, t, 0)),
        ],
        out_shape=[
            jax.ShapeDtypeStruct((B, N, _KPAD), jnp.int32),
            jax.ShapeDtypeStruct((B, N, HID), f32),
            jax.ShapeDtypeStruct((B, N, HID), f32),
        ],
    )(yT, sc1, sh1, wdT, wjT)

    agg, s2p, ss2p = pl.pallas_call(
        _p3,
        grid=(B, NT),
        in_specs=[
            pl.BlockSpec((1, TN, _KPAD), lambda b, t: (b, t, 0)),
            pl.BlockSpec((1, N, HID), lambda b, t: (b, 0, 0)),
            pl.BlockSpec((1, TN, HID), lambda b, t: (b, t, 0)),
            pl.BlockSpec((1, HID), lambda b, t: (0, 0)),
        ],
        out_specs=[
            pl.BlockSpec((1, TN, HID), lambda b, t: (b, t, 0)),
            pl.BlockSpec((1, 1, HID), lambda b, t: (b * NT + t, 0, 0)),
            pl.BlockSpec((1, 1, HID), lambda b, t: (b * NT + t, 0, 0)),
        ],
        out_shape=[
            jax.ShapeDtypeStruct((B, N, HID), f32),
            jax.ShapeDtypeStruct((B * NT, 1, HID), f32),
            jax.ShapeDtypeStruct((B * NT, 1, HID), f32),
        ],
    )(idx, bfT, aT, b_gc[None, :])
    sc2, sh2 = _affine(s2p, ss2p, g2, be2, B * N)

    outp, s3p, ss3p = pl.pallas_call(
        _p5,
        grid=(B, NT),
        in_specs=[
            pl.BlockSpec((1, TN, HID), lambda b, t: (b, t, 0)),
            pl.BlockSpec((1, HID), lambda b, t: (0, 0)),
            pl.BlockSpec((1, HID), lambda b, t: (0, 0)),
            pl.BlockSpec((HID, C), lambda b, t: (0, 0)),
            pl.BlockSpec((1, C), lambda b, t: (0, 0)),
        ],
        out_specs=[
            pl.BlockSpec((1, TN, C), lambda b, t: (b, t, 0)),
            pl.BlockSpec((1, 1, C), lambda b, t: (b * NT + t, 0, 0)),
            pl.BlockSpec((1, 1, C), lambda b, t: (b * NT + t, 0, 0)),
        ],
        out_shape=[
            jax.ShapeDtypeStruct((B, N, C), f32),
            jax.ShapeDtypeStruct((B * NT, 1, C), f32),
            jax.ShapeDtypeStruct((B * NT, 1, C), f32),
        ],
    )(agg, sc2, sh2, w2t, b_fc2[None, :])
    sc3, sh3 = _affine(s3p, ss3p, g3, be3, B * N)

    outT = pl.pallas_call(
        _p6,
        grid=(B,),
        in_specs=[
            pl.BlockSpec((1, N, C), lambda b: (b, 0, 0)),
            pl.BlockSpec((1, C), lambda b: (0, 0)),
            pl.BlockSpec((1, C), lambda b: (0, 0)),
            pl.BlockSpec((1, N, C), lambda b: (b, 0, 0)),
        ],
        out_specs=pl.BlockSpec((1, N, C), lambda b: (b, 0, 0)),
        out_shape=jax.ShapeDtypeStruct((B, N, C), f32),
    )(outp, sc3, sh3, xT)

    return jnp.transpose(outT, (0, 2, 1)).reshape(B, C, H, Wd)
